# baseline (speedup 1.0000x reference)
import functools

import jax
import jax.numpy as jnp
from jax.experimental import pallas as pl
from jax.experimental.pallas import tpu as pltpu

_M_BLK = 16
_NBUF = 6


def _body(x_hbm, wt_ref, o_ref, *scratch, m):
    bufs = scratch[:_NBUF]
    sems = scratch[_NBUF]
    npanels = m // _M_BLK
    rounds = npanels // _NBUF
    leftover = npanels - rounds * _NBUF

    def copy(i, s):
        return pltpu.make_async_copy(
            x_hbm.at[pl.ds(i * _M_BLK, _M_BLK), :], bufs[s], sems.at[s]
        )

    for s in range(min(_NBUF, npanels)):
        copy(s, s).start()

    def process(i, s):
        copy(i, s).wait()
        part = jax.lax.dot_general(
            bufs[s][...], wt_ref[...], (((1,), (1,)), ((), ())),
            preferred_element_type=jnp.float32,
        )
        o_ref[pl.ds(i * _M_BLK, _M_BLK), :] = part

    def round_body(r, carry):
        for s in range(_NBUF):
            i = r * _NBUF + s
            process(i, s)
            nxt = i + _NBUF

            @pl.when(nxt < npanels)
            def _refill():
                copy(nxt, s).start()
        return carry

    jax.lax.fori_loop(0, rounds, round_body, 0, unroll=False)
    for s in range(leftover):
        process(rounds * _NBUF + s, s)


def kernel(x, W):
    m, k_total = x.shape
    _, n = W.shape
    wt = W.T
    return pl.pallas_call(
        functools.partial(_body, m=m),
        in_specs=[
            pl.BlockSpec(memory_space=pltpu.MemorySpace.HBM),
            pl.BlockSpec((n, k_total), lambda: (0, 0)),
        ],
        out_specs=pl.BlockSpec(memory_space=pltpu.MemorySpace.VMEM),
        out_shape=jax.ShapeDtypeStruct((m, n), jnp.float32),
        scratch_shapes=[pltpu.VMEM((_M_BLK, k_total), jnp.float32)
                        for _ in range(_NBUF)]
        + [pltpu.SemaphoreType.DMA((_NBUF,))],
    )(x, wt)


# speedup vs baseline: 4.1214x; 4.1214x over previous
import functools

import jax
import jax.numpy as jnp
from jax.experimental import pallas as pl
from jax.experimental.pallas import tpu as pltpu

_K_BLK = 2048


def _mm_body(xt_ref, wt_ref, o_ref, *, k_total, nk):
    k = pl.program_id(0)

    @pl.when(k == 0)
    def _init():
        o_ref[...] = jnp.zeros_like(o_ref)

    def contract(xb):
        return jax.lax.dot_general(
            xb, wt_ref[...], (((0,), (1,)), ((), ())),
            preferred_element_type=jnp.float32,
        )

    @pl.when(k < nk - 1)
    def _full():
        o_ref[...] += contract(xt_ref[...])

    @pl.when(k == nk - 1)
    def _tail():
        rem = k_total - (nk - 1) * _K_BLK
        xb = xt_ref[...]
        row = jax.lax.broadcasted_iota(jnp.int32, xb.shape, 0)
        o_ref[...] += contract(jnp.where(row < rem, xb, 0.0))


def kernel(x, W):
    m, k_total = x.shape
    _, n = W.shape
    nk = pl.cdiv(k_total, _K_BLK)
    xt = jnp.transpose(x)
    wt = jnp.pad(jnp.transpose(W), ((0, 0), (0, nk * _K_BLK - k_total)))
    return pl.pallas_call(
        functools.partial(_mm_body, k_total=k_total, nk=nk),
        grid=(nk,),
        in_specs=[
            pl.BlockSpec((_K_BLK, m), lambda k: (k, 0)),
            pl.BlockSpec((n, _K_BLK), lambda k: (0, k)),
        ],
        out_specs=pl.BlockSpec((m, n), lambda k: (0, 0)),
        out_shape=jax.ShapeDtypeStruct((m, n), jnp.float32),
        compiler_params=pltpu.CompilerParams(
            dimension_semantics=("arbitrary",),
        ),
    )(xt, wt)


# speedup vs baseline: 4.2022x; 1.0196x over previous
import functools

import jax
import jax.numpy as jnp
from jax.experimental import pallas as pl
from jax.experimental.pallas import tpu as pltpu

_K_BLK = 4096


def _mm_body(xt_ref, wt_ref, o_ref, *, k_total, nk):
    k = pl.program_id(0)

    @pl.when(k == 0)
    def _init():
        o_ref[...] = jnp.zeros_like(o_ref)

    def contract(xb):
        return jax.lax.dot_general(
            xb, wt_ref[...], (((0,), (1,)), ((), ())),
            preferred_element_type=jnp.float32,
        )

    @pl.when(k < nk - 1)
    def _full():
        o_ref[...] += contract(xt_ref[...])

    @pl.when(k == nk - 1)
    def _tail():
        rem = k_total - (nk - 1) * _K_BLK
        xb = xt_ref[...]
        row = jax.lax.broadcasted_iota(jnp.int32, xb.shape, 0)
        o_ref[...] += contract(jnp.where(row < rem, xb, 0.0))


def kernel(x, W):
    m, k_total = x.shape
    _, n = W.shape
    nk = pl.cdiv(k_total, _K_BLK)
    xt = jnp.transpose(x)
    wt = jnp.pad(jnp.transpose(W), ((0, 0), (0, nk * _K_BLK - k_total)))
    return pl.pallas_call(
        functools.partial(_mm_body, k_total=k_total, nk=nk),
        grid=(nk,),
        in_specs=[
            pl.BlockSpec((_K_BLK, m), lambda k: (k, 0)),
            pl.BlockSpec((n, _K_BLK), lambda k: (0, k)),
        ],
        out_specs=pl.BlockSpec((m, n), lambda k: (0, 0)),
        out_shape=jax.ShapeDtypeStruct((m, n), jnp.float32),
        compiler_params=pltpu.CompilerParams(
            dimension_semantics=("arbitrary",),
        ),
    )(xt, wt)
